# Initial kernel scaffold; baseline (speedup 1.0000x reference)
import jax
import jax.numpy as jnp
from jax.experimental import pallas as pl


def kernel(x, Wr, br, Wl, bl):
    raise NotImplementedError("write your pallas kernel here")


# speedup vs baseline: 1.9588x; 1.9588x over previous
import functools

import jax
import jax.numpy as jnp
from jax import lax
from jax.experimental import pallas as pl
from jax.experimental.pallas import tpu as pltpu
from jax.experimental.pallas import tpu_sc as plsc


def _router_body(x_ref, wr_ref, w_ref):
    w_ref[0, 0, 0] = jnp.sum(x_ref[...] * wr_ref[...])


def _router_scores(x3, wr2):
    bt, c, hw = x3.shape
    out = pl.pallas_call(
        _router_body,
        grid=(bt,),
        in_specs=[
            pl.BlockSpec((1, c, hw), lambda i: (i, 0, 0)),
            pl.BlockSpec((c, hw), lambda i: (0, 0)),
        ],
        out_specs=pl.BlockSpec((1, 1, 1), lambda i: (i, 0, 0),
                               memory_space=pltpu.SMEM),
        out_shape=jax.ShapeDtypeStruct((bt, 1, 1), jnp.float32),
    )(x3, wr2)
    return out


def _make_route(b, t, k, nc):
    def _route_body(w_hbm, a_hbm, b_hbm, wv, av, bv, wg, miv):
        wid = lax.axis_index("s") * nc + lax.axis_index("c")

        @pl.when(wid < b)
        def _():
            pltpu.sync_copy(w_hbm.at[wid], wv)
            w = wv[...]
            wg[pl.ds(t, t)] = w
            iot = lax.iota(jnp.int32, t)
            cnt = jnp.zeros(t, jnp.int32)
            for s in range(t):
                idx = jnp.full((t,), t + s, jnp.int32)
                ws = plsc.load_gather(wg, [idx])
                cnt = cnt + (ws > w).astype(jnp.int32)
            mask = cnt < k
            mi = mask.astype(jnp.int32)
            miv[pl.ds(t, t)] = mi
            cs = jnp.zeros(t, jnp.int32)
            for s in range(t):
                idx = jnp.full((t,), t + s, jnp.int32)
                ms = plsc.load_gather(miv, [idx])
                cs = cs + ms * (iot >= s).astype(jnp.int32)
            sel = jnp.logical_and(mask, cs <= k)
            av[...] = sel.astype(jnp.float32)
            bv[...] = jnp.logical_not(mask).astype(jnp.float32)
            pltpu.sync_copy(av, a_hbm.at[wid])
            pltpu.sync_copy(bv, b_hbm.at[wid])

    route = pl.kernel(
        _route_body,
        mesh=plsc.VectorSubcoreMesh(core_axis_name="c", subcore_axis_name="s"),
        out_type=[
            jax.ShapeDtypeStruct((b, t), jnp.float32),
            jax.ShapeDtypeStruct((b, t), jnp.float32),
        ],
        scratch_types=[
            pltpu.VMEM((t,), jnp.float32),
            pltpu.VMEM((t,), jnp.float32),
            pltpu.VMEM((t,), jnp.float32),
            pltpu.VMEM((2 * t,), jnp.float32),
            pltpu.VMEM((2 * t,), jnp.int32),
        ],
        compiler_params=pltpu.CompilerParams(needs_layout_passes=False),
    )
    return route


def _apply_body(a_ref, b_ref, x_ref, wl_ref, bl_ref, out_ref):
    i = pl.program_id(0)
    af = a_ref[i]
    bf = b_ref[i]
    xm = x_ref[0]
    y = lax.dot_general(
        wl_ref[...], xm,
        dimension_numbers=(((0,), (0,)), ((), ())),
        preferred_element_type=jnp.float32,
        precision=lax.Precision.HIGHEST,
    )
    out_ref[0] = af * (y + bl_ref[...]) + bf * xm


def _apply_pass(a1, b1, x3, wl, bl2):
    bt, c, hw = x3.shape
    out = pl.pallas_call(
        _apply_body,
        grid=(bt,),
        in_specs=[
            pl.BlockSpec(memory_space=pltpu.SMEM),
            pl.BlockSpec(memory_space=pltpu.SMEM),
            pl.BlockSpec((1, c, hw), lambda i: (i, 0, 0)),
            pl.BlockSpec((c, c), lambda i: (0, 0)),
            pl.BlockSpec((c, 1), lambda i: (0, 0)),
        ],
        out_specs=pl.BlockSpec((1, c, hw), lambda i: (i, 0, 0)),
        out_shape=jax.ShapeDtypeStruct((bt, c, hw), jnp.float32),
    )(a1, b1, x3, wl, bl2)
    return out


def kernel(x, Wr, br, Wl, bl):
    b, t, c, h, w = x.shape
    hw = h * w
    k = max(1, int(0.35 * t))
    x3 = x.reshape(b * t, c, hw)
    wr2 = Wr.reshape(c, hw)
    scores = _router_scores(x3, wr2).reshape(b, t)
    info = plsc.get_sparse_core_info()
    a2, b2 = _make_route(b, t, k, info.num_cores)(scores)
    out3 = _apply_pass(a2.reshape(b * t), b2.reshape(b * t), x3, Wl,
                       bl.reshape(c, 1))
    return out3.reshape(b, t, c, h, w)
